# Initial kernel scaffold; baseline (speedup 1.0000x reference)
import jax
import jax.numpy as jnp
from jax.experimental import pallas as pl


def kernel(gt_matches0, gt_matches1, scores, distance):
    raise NotImplementedError("write your pallas kernel here")


# speedup vs baseline: 4.4185x; 4.4185x over previous
import functools

import jax
import jax.numpy as jnp
from jax import lax
from jax.experimental import pallas as pl
from jax.experimental.pallas import tpu as pltpu
from jax.experimental.pallas import tpu_sc as plsc

_B, _N, _M = 4, 1024, 1024
_SROW = _M + 1
_GAMMA = 0.5
_LAMDA = 0.5

_NTILES = 32
_CHUNK = (_B * _N) // _NTILES


@functools.cache
def _make_sc_gather():
    mesh = plsc.VectorSubcoreMesh(core_axis_name="c", subcore_axis_name="s")
    return functools.partial(
        pl.kernel,
        mesh=mesh,
        out_type=jax.ShapeDtypeStruct((4, _B * _N), jnp.float32),
        scratch_types=[
            pltpu.VMEM((_CHUNK,), jnp.int32),
            pltpu.VMEM((_CHUNK,), jnp.int32),
            pltpu.VMEM((_CHUNK,), jnp.int32),
            pltpu.VMEM((_CHUNK,), jnp.float32),
            pltpu.SemaphoreType.DMA,
        ],
    )(_sc_gather_body)


def _sc_gather_body(scores_hbm, dist_hbm, gt0_hbm, gt1_hbm, out_hbm,
                    gt0_v, gt1_v, idx_v, val_v, sem):
    wid = lax.axis_index("s") * 2 + lax.axis_index("c")
    base_g = wid * _CHUNK
    b = base_g // _N
    p0 = base_g - b * _N
    pltpu.sync_copy(gt0_hbm.at[pl.ds(base_g, _CHUNK)], gt0_v)
    pltpu.sync_copy(gt1_hbm.at[pl.ds(base_g, _CHUNK)], gt1_v)
    sbase = b * (_SROW * _SROW)
    dbase = b * (_N * _M)

    def run_task(row, table_hbm, idx_fn):
        for k in range(_CHUNK // 16):
            sl = pl.ds(k * 16, 16)
            pos = p0 + k * 16 + lax.iota(jnp.int32, 16)
            idx_v[sl] = idx_fn(pos, gt0_v[sl], gt1_v[sl])
        pltpu.async_copy(table_hbm.at[idx_v], val_v, sem).wait()
        pltpu.sync_copy(val_v, out_hbm.at[jnp.int32(row), pl.ds(base_g, _CHUNK)])

    run_task(0, scores_hbm, lambda pos, g0, g1: sbase + pos * _SROW + g0)
    run_task(1, scores_hbm, lambda pos, g0, g1: sbase + g1 * _SROW + pos)
    run_task(2, dist_hbm, lambda pos, g0, g1: dbase + pos * _M + g0)
    run_task(3, dist_hbm, lambda pos, g0, g1: dbase + g1 * _M + pos)


def _i32(v):
    return jnp.int32(v)


_RB = 256
_NRB = 5
_KCNT = float(2 * _B * _N * (_M - 1))


def _tc_body(s_ref, d_ref, sp0_ref, dp0_ref, sp1_ref, dp1_ref, out_ref,
             colT, colC, colS, colQ, acc):
    b = pl.program_id(0)
    rb = pl.program_id(1)

    @pl.when(jnp.logical_and(b == 0, rb == 0))
    def _init_acc():
        acc[0] = 0.0
        acc[1] = 0.0
        acc[2] = 0.0
        acc[3] = 0.0
        acc[4] = 0.0

    @pl.when(rb == 0)
    def _init_cols():
        z = jnp.zeros((1, _M), jnp.float32)
        colT[...] = z
        colC[...] = z
        colS[...] = z
        colQ[...] = z
        acc[4] += jnp.sum(sp1_ref[0])

    S = s_ref[0]
    rowid = rb * _RB + lax.broadcasted_iota(jnp.int32, (_RB, 1), 0)

    sp0 = sp0_ref[0]
    x = (S - sp0) + _GAMMA
    T0 = jnp.sum(jnp.maximum(x, 0.0), axis=1, keepdims=True)
    C0 = jnp.sum((x > 0.0).astype(jnp.float32), axis=1, keepdims=True)
    rowterm = (T0 - _GAMMA) / jnp.maximum(C0 - 1.0, 1.0)
    acc[0] += jnp.sum(jnp.where(rowid < _N, rowterm, 0.0))

    sp1 = sp1_ref[0]
    y = (S[:, :_M] - sp1) + _GAMMA
    vmask = rowid < _N + 1
    colT[...] += jnp.sum(jnp.where(vmask, jnp.maximum(y, 0.0), 0.0),
                         axis=0, keepdims=True)
    colC[...] += jnp.sum(jnp.where(vmask, (y > 0.0).astype(jnp.float32), 0.0),
                         axis=0, keepdims=True)

    @pl.when(rb < _NRB - 1)
    def _dist():
        D = d_ref[0]
        D2 = D * D
        dp0 = dp0_ref[0]
        RS = jnp.sum(D, axis=1, keepdims=True)
        RQ = jnp.sum(D2, axis=1, keepdims=True)
        acc[2] += jnp.sum(float(_M) * dp0 - RS)
        acc[3] += jnp.sum(float(_M) * dp0 * dp0 - 2.0 * dp0 * RS + RQ)
        colS[...] += jnp.sum(D, axis=0, keepdims=True)
        colQ[...] += jnp.sum(D2, axis=0, keepdims=True)

    @pl.when(rb == _NRB - 1)
    def _fin_cols():
        colterm = (colT[...] - _GAMMA) / jnp.maximum(colC[...] - 1.0, 1.0)
        acc[1] += jnp.sum(colterm)
        dp1 = dp1_ref[0]
        CS = colS[...]
        CQ = colQ[...]
        acc[2] += jnp.sum(float(_N) * dp1 - CS)
        acc[3] += jnp.sum(float(_N) * dp1 * dp1 - 2.0 * dp1 * CS + CQ)

    @pl.when(jnp.logical_and(b == _B - 1, rb == _NRB - 1))
    def _final():
        denom = float(_B * _N)
        gap_total = (acc[0] / denom + acc[1] / denom) * 0.5
        ot_loss = -acc[4] / denom
        mean_margin = acc[2] / _KCNT
        var_loss = (acc[3] - acc[2] * acc[2] / _KCNT) / (_KCNT - 1.0)
        aml = jnp.exp(mean_margin)
        loss = ((ot_loss + aml + var_loss) * (1.0 - _LAMDA)
                + (gap_total + var_loss) * _LAMDA)
        out_ref[...] = jnp.reshape(loss, (1, 1))


_tc_call_kwargs = dict(
    grid=(_B, _NRB),
    in_specs=[
        pl.BlockSpec((1, _RB, _SROW), lambda b, rb: (b, rb, _i32(0))),
        pl.BlockSpec((1, _RB, _M),
                     lambda b, rb: (b, jnp.minimum(rb, _NRB - 2).astype(jnp.int32),
                                    _i32(0))),
        pl.BlockSpec((1, _RB, 1), lambda b, rb: (b, rb, _i32(0))),
        pl.BlockSpec((1, _RB, 1), lambda b, rb: (b, rb, _i32(0))),
        pl.BlockSpec((1, 1, _M), lambda b, rb: (b, _i32(0), _i32(0))),
        pl.BlockSpec((1, 1, _M), lambda b, rb: (b, _i32(0), _i32(0))),
    ],
    out_specs=pl.BlockSpec((1, 1), lambda b, rb: (_i32(0), _i32(0))),
    out_shape=jax.ShapeDtypeStruct((1, 1), jnp.float32),
    scratch_shapes=[
        pltpu.VMEM((1, _M), jnp.float32),
        pltpu.VMEM((1, _M), jnp.float32),
        pltpu.VMEM((1, _M), jnp.float32),
        pltpu.VMEM((1, _M), jnp.float32),
        pltpu.SMEM((8,), jnp.float32),
    ],
    compiler_params=pltpu.CompilerParams(
        dimension_semantics=("arbitrary", "arbitrary")),
)


@functools.cache
def _make_tc_call():
    return pl.pallas_call(_tc_body, **_tc_call_kwargs)


def kernel(gt_matches0, gt_matches1, scores, distance):
    scores = scores.astype(jnp.float32)
    distance = distance.astype(jnp.float32)
    gt0f = gt_matches0.astype(jnp.int32).reshape(-1)
    gt1f = gt_matches1.astype(jnp.int32).reshape(-1)

    gathered = _make_sc_gather()(scores.reshape(-1), distance.reshape(-1),
                                 gt0f, gt1f)
    pad = _NRB * _RB - _N
    sp0 = jnp.pad(gathered[0].reshape(_B, _N), ((0, 0), (0, pad)))[..., None]
    dp0 = jnp.pad(gathered[2].reshape(_B, _N), ((0, 0), (0, pad)))[..., None]
    sp1 = gathered[1].reshape(_B, 1, _N)
    dp1 = gathered[3].reshape(_B, 1, _N)

    out = _make_tc_call()(scores, distance, sp0, dp0, sp1, dp1)
    return out[0, 0]


# speedup vs baseline: 16.9024x; 3.8254x over previous
import functools

import jax
import jax.numpy as jnp
from jax import lax
from jax.experimental import pallas as pl
from jax.experimental.pallas import tpu as pltpu

_B, _N, _M = 4, 1024, 1024
_SROW = _M + 1
_GAMMA = 0.5
_LAMDA = 0.5

_RB = 256
_NRB = 5
_CB = 256
_NCB = 4
_NSTEP = _NRB + _NCB
_KCNT = float(2 * _B * _N * (_M - 1))


def _i32(v):
    return jnp.int32(v)


def _body(srow_ref, drow_ref, scol_ref, dcol_ref, gt0_ref, gt1_ref, out_ref,
          acc):
    b = pl.program_id(0)
    step = pl.program_id(1)

    @pl.when(jnp.logical_and(b == 0, step == 0))
    def _init_acc():
        acc[0] = 0.0
        acc[1] = 0.0
        acc[2] = 0.0
        acc[3] = 0.0
        acc[4] = 0.0

    @pl.when(step < _NRB)
    def _phase_a():
        S = srow_ref[0]
        gt0 = gt0_ref[0]
        rowid = step * _RB + lax.broadcasted_iota(jnp.int32, (_RB, 1), 0)
        cid = lax.broadcasted_iota(jnp.int32, (_RB, _SROW), 1)
        onehot = cid == gt0
        s_pos0 = jnp.sum(jnp.where(onehot, S, 0.0), axis=1, keepdims=True)
        x = (S - s_pos0) + _GAMMA
        T0 = jnp.sum(jnp.maximum(x, 0.0), axis=1, keepdims=True)
        C0 = jnp.sum((x > 0.0).astype(jnp.float32), axis=1, keepdims=True)
        rowterm = (T0 - _GAMMA) / jnp.maximum(C0 - 1.0, 1.0)
        acc[0] += jnp.sum(jnp.where(rowid < _N, rowterm, 0.0))

        @pl.when(step < _NRB - 1)
        def _dist_rows():
            D = drow_ref[0]
            d_pos0 = jnp.sum(jnp.where(onehot[:, :_M], D, 0.0),
                             axis=1, keepdims=True)
            RS = jnp.sum(D, axis=1, keepdims=True)
            RQ = jnp.sum(D * D, axis=1, keepdims=True)
            acc[2] += jnp.sum(float(_M) * d_pos0 - RS)
            acc[3] += jnp.sum(float(_M) * d_pos0 * d_pos0
                              - 2.0 * d_pos0 * RS + RQ)

    @pl.when(step >= _NRB)
    def _phase_b():
        Sc = scol_ref[0]
        Dc = dcol_ref[0]
        gt1 = gt1_ref[0]
        rid_s = lax.broadcasted_iota(jnp.int32, (_SROW, _CB), 0)
        onehot1 = rid_s == gt1
        s_pos1 = jnp.sum(jnp.where(onehot1, Sc, 0.0), axis=0, keepdims=True)
        y = (Sc - s_pos1) + _GAMMA
        T1 = jnp.sum(jnp.maximum(y, 0.0), axis=0, keepdims=True)
        C1 = jnp.sum((y > 0.0).astype(jnp.float32), axis=0, keepdims=True)
        colterm = (T1 - _GAMMA) / jnp.maximum(C1 - 1.0, 1.0)
        acc[1] += jnp.sum(colterm)
        acc[4] += jnp.sum(s_pos1)

        d_pos1 = jnp.sum(jnp.where(onehot1[:_N, :], Dc, 0.0),
                         axis=0, keepdims=True)
        CS = jnp.sum(Dc, axis=0, keepdims=True)
        CQ = jnp.sum(Dc * Dc, axis=0, keepdims=True)
        acc[2] += jnp.sum(float(_N) * d_pos1 - CS)
        acc[3] += jnp.sum(float(_N) * d_pos1 * d_pos1 - 2.0 * d_pos1 * CS + CQ)

    @pl.when(jnp.logical_and(b == _B - 1, step == _NSTEP - 1))
    def _final():
        denom = float(_B * _N)
        gap_total = (acc[0] / denom + acc[1] / denom) * 0.5
        ot_loss = -acc[4] / denom
        mean_margin = acc[2] / _KCNT
        var_loss = (acc[3] - acc[2] * acc[2] / _KCNT) / (_KCNT - 1.0)
        aml = jnp.exp(mean_margin)
        loss = ((ot_loss + aml + var_loss) * (1.0 - _LAMDA)
                + (gap_total + var_loss) * _LAMDA)
        out_ref[...] = jnp.reshape(loss, (1, 1))


def _clamp_col(step):
    return jnp.clip(step - _NRB, 0, _NCB - 1).astype(jnp.int32)


_call_kwargs = dict(
    grid=(_B, _NSTEP),
    in_specs=[
        pl.BlockSpec((1, _RB, _SROW),
                     lambda b, s: (b, jnp.minimum(s, _NRB - 1).astype(jnp.int32),
                                   _i32(0))),
        pl.BlockSpec((1, _RB, _M),
                     lambda b, s: (b, jnp.minimum(s, _NRB - 2).astype(jnp.int32),
                                   _i32(0))),
        pl.BlockSpec((1, _SROW, _CB), lambda b, s: (b, _i32(0), _clamp_col(s))),
        pl.BlockSpec((1, _N, _CB), lambda b, s: (b, _i32(0), _clamp_col(s))),
        pl.BlockSpec((1, _RB, 1),
                     lambda b, s: (b, jnp.minimum(s, _NRB - 1).astype(jnp.int32),
                                   _i32(0))),
        pl.BlockSpec((1, 1, _CB), lambda b, s: (b, _i32(0), _clamp_col(s))),
    ],
    out_specs=pl.BlockSpec((1, 1), lambda b, s: (_i32(0), _i32(0))),
    out_shape=jax.ShapeDtypeStruct((1, 1), jnp.float32),
    scratch_shapes=[pltpu.SMEM((8,), jnp.float32)],
    compiler_params=pltpu.CompilerParams(
        dimension_semantics=("arbitrary", "arbitrary")),
)


@functools.cache
def _make_call():
    return pl.pallas_call(_body, **_call_kwargs)


def kernel(gt_matches0, gt_matches1, scores, distance):
    scores = scores.astype(jnp.float32)
    distance = distance.astype(jnp.float32)
    pad = _NRB * _RB - _N
    gt0 = jnp.pad(gt_matches0.astype(jnp.int32), ((0, 0), (0, pad)),
                  constant_values=-1)[..., None]
    gt1 = gt_matches1.astype(jnp.int32)[:, None, :]

    out = _make_call()(scores, distance, scores, distance, gt0, gt1)
    return out[0, 0]
